# Initial kernel scaffold; baseline (speedup 1.0000x reference)
import jax
import jax.numpy as jnp
from jax.experimental import pallas as pl


def kernel(x_nchw, conv1_w, conv1_bss, conv2_w, conv2_bss, conv3_w, conv3_bss, conv4_w, conv4_bss, fc5_w, fc5_bss, fc1_w, fc1_b, fc2_w, fc2_b):
    raise NotImplementedError("write your pallas kernel here")


# speedup vs baseline: 1.6874x; 1.6874x over previous
import functools

import jax
import jax.numpy as jnp
from jax.experimental import pallas as pl
from jax.experimental.pallas import tpu as pltpu

_VMEM_LIMIT = 48 * 1024 * 1024


def _cp(dims):
    return pltpu.CompilerParams(dimension_semantics=dims,
                                vmem_limit_bytes=_VMEM_LIMIT)


def _affine(y, bss, relu_first):
    y = y + bss[0:1, :]
    if relu_first:
        return jnp.maximum(y, 0.0) * bss[1:2, :] + bss[2:3, :]
    return jnp.maximum(y * bss[1:2, :] + bss[2:3, :], 0.0)


def _pool2x2(y, H, W, C):
    y = y.reshape(H, W // 2, 2 * C)
    y = y.reshape(H // 2, 2, W // 2, 2 * C)
    m = jnp.maximum(y[:, 0], y[:, 1])
    return jnp.maximum(m[:, :, :C], m[:, :, C:])


def _conv_taps(p, w_ref, H, W, k):
    cin = w_ref.shape[1]
    cout = w_ref.shape[2]
    acc = jnp.zeros((H * W, cout), jnp.float32)
    for s in range(k * k):
        i, j = divmod(s, k)
        v = p[i:i + H, j:j + W, :].reshape(H * W, cin)
        acc = acc + jnp.dot(v, w_ref[s], preferred_element_type=jnp.float32)
    return acc


def _extractor_kernel(x_ref, w1_ref, bss1_ref, w2_ref, bss2_ref,
                      w3_ref, bss3_ref, w4_ref, bss4_ref, o_ref):
    x = x_ref[0]
    pw = jnp.concatenate([x[:, j:j + 64, :] for j in range(5)], axis=-1)
    acc = jnp.zeros((4096, 128), jnp.float32)
    for i in range(5):
        q = pw[i:i + 64].reshape(4096, 20)
        acc = acc + jnp.dot(q, w1_ref[i], preferred_element_type=jnp.float32)
    y = _affine(acc, bss1_ref, True)
    y = _pool2x2(y, 64, 64, 128).astype(jnp.bfloat16)

    p = jnp.pad(y, ((2, 2), (2, 2), (0, 0)))
    y = _affine(_conv_taps(p, w2_ref, 32, 32, 5), bss2_ref, True)
    y = _pool2x2(y, 32, 32, 128).astype(jnp.bfloat16)

    p = jnp.pad(y, ((1, 1), (1, 1), (0, 0)))
    y = _affine(_conv_taps(p, w3_ref, 16, 16, 3), bss3_ref, True)
    y = _pool2x2(y, 16, 16, 256).astype(jnp.bfloat16)

    p = jnp.pad(y, ((1, 1), (1, 1), (0, 0)))
    y = _affine(_conv_taps(p, w4_ref, 8, 8, 3), bss4_ref, False)
    o_ref[0] = y.astype(jnp.bfloat16)


def _extract(xp, w1m, bss1, w2, bss2, w3, bss3, w4, bss4):
    n = xp.shape[0]
    return pl.pallas_call(
        _extractor_kernel,
        out_shape=jax.ShapeDtypeStruct((n, 64, 512), jnp.bfloat16),
        grid=(n,),
        in_specs=[
            pl.BlockSpec((1, 68, 68, 4), lambda b: (b, 0, 0, 0)),
            pl.BlockSpec((5, 20, 128), lambda b: (0, 0, 0)),
            pl.BlockSpec((3, 128), lambda b: (0, 0)),
            pl.BlockSpec((25, 128, 128), lambda b: (0, 0, 0)),
            pl.BlockSpec((3, 128), lambda b: (0, 0)),
            pl.BlockSpec((9, 128, 256), lambda b: (0, 0, 0)),
            pl.BlockSpec((3, 256), lambda b: (0, 0)),
            pl.BlockSpec((9, 256, 512), lambda b: (0, 0, 0)),
            pl.BlockSpec((3, 512), lambda b: (0, 0)),
        ],
        out_specs=pl.BlockSpec((1, 64, 512), lambda b: (b, 0, 0)),
        compiler_params=_cp(("parallel",)),
    )(xp, w1m, bss1, w2, bss2, w3, bss3, w4, bss4)


def _fc5_kernel(f_ref, w_ref, bss_ref, o_ref, acc_ref):
    k = pl.program_id(1)

    @pl.when(k == 0)
    def _init():
        acc_ref[...] = jnp.zeros_like(acc_ref)

    acc_ref[...] += jnp.dot(f_ref[...], w_ref[...],
                            preferred_element_type=jnp.float32)

    @pl.when(k == pl.num_programs(1) - 1)
    def _finish():
        o_ref[...] = _affine(acc_ref[...], bss_ref[...], True
                             ).astype(jnp.bfloat16)


def _fc5(flat, fc5_w, fc5_bss, tk=4096, tn=512):
    m, k5 = flat.shape
    n5 = fc5_w.shape[1]
    grid = (n5 // tn, k5 // tk)
    return pl.pallas_call(
        _fc5_kernel,
        out_shape=jax.ShapeDtypeStruct((m, n5), jnp.bfloat16),
        grid=grid,
        in_specs=[
            pl.BlockSpec((m, tk), lambda n, k: (0, k)),
            pl.BlockSpec((tk, tn), lambda n, k: (k, n)),
            pl.BlockSpec((3, tn), lambda n, k: (0, n)),
        ],
        out_specs=pl.BlockSpec((m, tn), lambda n, k: (0, n)),
        scratch_shapes=[pltpu.VMEM((m, tn), jnp.float32)],
        compiler_params=_cp(("parallel", "arbitrary")),
    )(flat, fc5_w, fc5_bss)


def _mlp_kernel(fv_ref, w1_ref, b1_ref, w2_ref, b2_ref, o_ref, *, batch):
    fv = fv_ref[...]
    n5 = fv.shape[1]
    h = jnp.dot(fv[:batch], w1_ref[:n5], preferred_element_type=jnp.float32)
    h = h + jnp.dot(fv[batch:], w1_ref[n5:],
                    preferred_element_type=jnp.float32)
    h = jnp.maximum(h + b1_ref[...], 0.0).astype(jnp.bfloat16)
    o_ref[...] = jnp.dot(h, w2_ref[...],
                         preferred_element_type=jnp.float32) + b2_ref[...]


def _mlp(fv, fc1_w, fc1_b, fc2_w, fc2_b, batch):
    return pl.pallas_call(
        functools.partial(_mlp_kernel, batch=batch),
        out_shape=jax.ShapeDtypeStruct((batch, 2), jnp.float32),
        grid=(1,),
        in_specs=[
            pl.BlockSpec(fv.shape, lambda i: (0, 0)),
            pl.BlockSpec(fc1_w.shape, lambda i: (0, 0)),
            pl.BlockSpec(fc1_b.shape, lambda i: (0, 0)),
            pl.BlockSpec(fc2_w.shape, lambda i: (0, 0)),
            pl.BlockSpec(fc2_b.shape, lambda i: (0, 0)),
        ],
        out_specs=pl.BlockSpec((batch, 2), lambda i: (0, 0)),
        compiler_params=_cp(("arbitrary",)),
    )(fv, fc1_w, fc1_b, fc2_w, fc2_b)


def kernel(x_nchw, conv1_w, conv1_bss, conv2_w, conv2_bss, conv3_w, conv3_bss,
           conv4_w, conv4_bss, fc5_w, fc5_bss, fc1_w, fc1_b, fc2_w, fc2_b):
    B = x_nchw.shape[0]
    xb = jnp.concatenate([x_nchw[:, 0:3], x_nchw[:, 3:6]], axis=0)
    xb = jnp.transpose(xb, (0, 2, 3, 1)).astype(jnp.bfloat16)
    xp = jnp.pad(xb, ((0, 0), (2, 2), (2, 2), (0, 1)))

    w1 = conv1_w.reshape(5, 5, 3, 128)
    w1 = jnp.pad(w1, ((0, 0), (0, 0), (0, 1), (0, 0))).reshape(5, 20, 128)

    feats = _extract(xp, w1, conv1_bss, conv2_w, conv2_bss,
                     conv3_w, conv3_bss, conv4_w, conv4_bss)
    flat = feats.reshape(2 * B, 64 * 512)
    fv = _fc5(flat, fc5_w, fc5_bss)
    return _mlp(fv, fc1_w, fc1_b, fc2_w, fc2_b, B)


# speedup vs baseline: 2.1695x; 1.2857x over previous
import functools

import jax
import jax.numpy as jnp
from jax.experimental import pallas as pl
from jax.experimental.pallas import tpu as pltpu

_VMEM_LIMIT = 48 * 1024 * 1024


def _cp(dims):
    return pltpu.CompilerParams(dimension_semantics=dims,
                                vmem_limit_bytes=_VMEM_LIMIT)


def _affine(y, bss, relu_first):
    y = y + bss[0:1, :]
    if relu_first:
        return jnp.maximum(y, 0.0) * bss[1:2, :] + bss[2:3, :]
    return jnp.maximum(y * bss[1:2, :] + bss[2:3, :], 0.0)


def _pool2x2(y, H, W, C):
    y = y.reshape(H, W // 2, 2 * C)
    y = y.reshape(H // 2, 2, W // 2, 2 * C)
    m = jnp.maximum(y[:, 0], y[:, 1])
    return jnp.maximum(m[:, :, :C], m[:, :, C:])


def _conv_wmerge(p, w_ref, H, W, k):
    c = p.shape[2]
    cout = w_ref.shape[2]
    pw = jnp.concatenate([p[:, j:j + W, :] for j in range(k)], axis=-1)
    acc = jnp.zeros((H * W, cout), jnp.float32)
    for i in range(k):
        q = pw[i:i + H].reshape(H * W, k * c)
        acc = acc + jnp.dot(q, w_ref[i], preferred_element_type=jnp.float32)
    return acc


def _extractor_kernel(x_ref, w1_ref, bss1_ref, w2_ref, bss2_ref,
                      w3_ref, bss3_ref, w4_ref, bss4_ref, o_ref):
    x = x_ref[0]
    pw = jnp.concatenate([x[:, j:j + 64, :] for j in range(5)], axis=-1)
    p = jnp.concatenate([pw[i:i + 64] for i in range(5)], axis=-1)
    acc = jnp.dot(p.reshape(4096, 100), w1_ref[...],
                  preferred_element_type=jnp.float32)
    y = _affine(acc, bss1_ref, True)
    y = _pool2x2(y, 64, 64, 128).astype(jnp.bfloat16)

    p = jnp.pad(y, ((2, 2), (2, 2), (0, 0)))
    y = _affine(_conv_wmerge(p, w2_ref, 32, 32, 5), bss2_ref, True)
    y = _pool2x2(y, 32, 32, 128).astype(jnp.bfloat16)

    p = jnp.pad(y, ((1, 1), (1, 1), (0, 0)))
    y = _affine(_conv_wmerge(p, w3_ref, 16, 16, 3), bss3_ref, True)
    y = _pool2x2(y, 16, 16, 256).astype(jnp.bfloat16)

    p = jnp.pad(y, ((1, 1), (1, 1), (0, 0)))
    y = _affine(_conv_wmerge(p, w4_ref, 8, 8, 3), bss4_ref, False)
    o_ref[0] = y.astype(jnp.bfloat16)


def _extract(xp, w1m, bss1, w2, bss2, w3, bss3, w4, bss4):
    n = xp.shape[0]
    return pl.pallas_call(
        _extractor_kernel,
        out_shape=jax.ShapeDtypeStruct((n, 64, 512), jnp.bfloat16),
        grid=(n,),
        in_specs=[
            pl.BlockSpec((1, 68, 68, 4), lambda b: (b, 0, 0, 0)),
            pl.BlockSpec((100, 128), lambda b: (0, 0)),
            pl.BlockSpec((3, 128), lambda b: (0, 0)),
            pl.BlockSpec((5, 640, 128), lambda b: (0, 0, 0)),
            pl.BlockSpec((3, 128), lambda b: (0, 0)),
            pl.BlockSpec((3, 384, 256), lambda b: (0, 0, 0)),
            pl.BlockSpec((3, 256), lambda b: (0, 0)),
            pl.BlockSpec((3, 768, 512), lambda b: (0, 0, 0)),
            pl.BlockSpec((3, 512), lambda b: (0, 0)),
        ],
        out_specs=pl.BlockSpec((1, 64, 512), lambda b: (b, 0, 0)),
        compiler_params=_cp(("parallel",)),
    )(xp, w1m, bss1, w2, bss2, w3, bss3, w4, bss4)


def _fc5_kernel(f_ref, w_ref, bss_ref, o_ref, acc_ref):
    k = pl.program_id(1)

    @pl.when(k == 0)
    def _init():
        acc_ref[...] = jnp.zeros_like(acc_ref)

    acc_ref[...] += jnp.dot(f_ref[...], w_ref[...],
                            preferred_element_type=jnp.float32)

    @pl.when(k == pl.num_programs(1) - 1)
    def _finish():
        o_ref[...] = _affine(acc_ref[...], bss_ref[...], True
                             ).astype(jnp.bfloat16)


def _fc5(flat, fc5_w, fc5_bss, tk=4096, tn=512):
    m, k5 = flat.shape
    n5 = fc5_w.shape[1]
    grid = (n5 // tn, k5 // tk)
    return pl.pallas_call(
        _fc5_kernel,
        out_shape=jax.ShapeDtypeStruct((m, n5), jnp.bfloat16),
        grid=grid,
        in_specs=[
            pl.BlockSpec((m, tk), lambda n, k: (0, k)),
            pl.BlockSpec((tk, tn), lambda n, k: (k, n)),
            pl.BlockSpec((3, tn), lambda n, k: (0, n)),
        ],
        out_specs=pl.BlockSpec((m, tn), lambda n, k: (0, n)),
        scratch_shapes=[pltpu.VMEM((m, tn), jnp.float32)],
        compiler_params=_cp(("parallel", "arbitrary")),
    )(flat, fc5_w, fc5_bss)


def _mlp_kernel(fv_ref, w1_ref, b1_ref, w2_ref, b2_ref, o_ref, *, batch):
    fv = fv_ref[...]
    n5 = fv.shape[1]
    h = jnp.dot(fv[:batch], w1_ref[:n5], preferred_element_type=jnp.float32)
    h = h + jnp.dot(fv[batch:], w1_ref[n5:],
                    preferred_element_type=jnp.float32)
    h = jnp.maximum(h + b1_ref[...], 0.0).astype(jnp.bfloat16)
    o_ref[...] = jnp.dot(h, w2_ref[...],
                         preferred_element_type=jnp.float32) + b2_ref[...]


def _mlp(fv, fc1_w, fc1_b, fc2_w, fc2_b, batch):
    return pl.pallas_call(
        functools.partial(_mlp_kernel, batch=batch),
        out_shape=jax.ShapeDtypeStruct((batch, 2), jnp.float32),
        grid=(1,),
        in_specs=[
            pl.BlockSpec(fv.shape, lambda i: (0, 0)),
            pl.BlockSpec(fc1_w.shape, lambda i: (0, 0)),
            pl.BlockSpec(fc1_b.shape, lambda i: (0, 0)),
            pl.BlockSpec(fc2_w.shape, lambda i: (0, 0)),
            pl.BlockSpec(fc2_b.shape, lambda i: (0, 0)),
        ],
        out_specs=pl.BlockSpec((batch, 2), lambda i: (0, 0)),
        compiler_params=_cp(("arbitrary",)),
    )(fv, fc1_w, fc1_b, fc2_w, fc2_b)


def kernel(x_nchw, conv1_w, conv1_bss, conv2_w, conv2_bss, conv3_w, conv3_bss,
           conv4_w, conv4_bss, fc5_w, fc5_bss, fc1_w, fc1_b, fc2_w, fc2_b):
    B = x_nchw.shape[0]
    xb = jnp.concatenate([x_nchw[:, 0:3], x_nchw[:, 3:6]], axis=0)
    xb = jnp.transpose(xb, (0, 2, 3, 1)).astype(jnp.bfloat16)
    xp = jnp.pad(xb, ((0, 0), (2, 2), (2, 2), (0, 1)))

    w1 = conv1_w.reshape(5, 5, 3, 128)
    w1 = jnp.pad(w1, ((0, 0), (0, 0), (0, 1), (0, 0))).reshape(100, 128)
    w2 = conv2_w.reshape(5, 5 * 128, 128)
    w3 = conv3_w.reshape(3, 3 * 128, 256)
    w4 = conv4_w.reshape(3, 3 * 256, 512)

    feats = _extract(xp, w1, conv1_bss, w2, conv2_bss,
                     w3, conv3_bss, w4, conv4_bss)
    flat = feats.reshape(2 * B, 64 * 512)
    fv = _fc5(flat, fc5_w, fc5_bss)
    return _mlp(fv, fc1_w, fc1_b, fc2_w, fc2_b, B)


# speedup vs baseline: 2.9855x; 1.3761x over previous
import functools

import jax
import jax.numpy as jnp
from jax.experimental import pallas as pl
from jax.experimental.pallas import tpu as pltpu

_VMEM_LIMIT = 48 * 1024 * 1024


def _cp(dims):
    return pltpu.CompilerParams(dimension_semantics=dims,
                                vmem_limit_bytes=_VMEM_LIMIT)


def _affine(y, bss, relu_first):
    y = y + bss[0:1, :]
    if relu_first:
        return jnp.maximum(y, 0.0) * bss[1:2, :] + bss[2:3, :]
    return jnp.maximum(y * bss[1:2, :] + bss[2:3, :], 0.0)


def _pool2x2(y, H, W, C):
    y = y.reshape(H, W // 2, 2 * C)
    y = y.reshape(H // 2, 2, W // 2, 2 * C)
    m = jnp.maximum(y[:, 0], y[:, 1])
    return jnp.maximum(m[:, :, :C], m[:, :, C:])


def _conv_wmerge(p, w_ref, H, W, k):
    c = p.shape[2]
    cout = w_ref.shape[2]
    pw = jnp.concatenate([p[:, j:j + W, :] for j in range(k)], axis=-1)
    acc = jnp.zeros((H * W, cout), jnp.float32)
    for i in range(k):
        q = pw[i:i + H].reshape(H * W, k * c)
        acc = acc + jnp.dot(q, w_ref[i], preferred_element_type=jnp.float32)
    return acc


def _extractor_kernel(x_ref, w1_ref, bss1_ref, w2_ref, bss2_ref,
                      w3_ref, bss3_ref, w4_ref, bss4_ref, o_ref):
    for u in range(x_ref.shape[1]):
        x = jnp.transpose(x_ref[0, u], (1, 2, 0))
        pw = jnp.concatenate([x[:, j:j + 64, :] for j in range(5)], axis=-1)
        p = jnp.concatenate([pw[i:i + 64] for i in range(5)], axis=-1)
        acc = jnp.dot(p.reshape(4096, 100), w1_ref[...],
                      preferred_element_type=jnp.float32)
        y = _affine(acc, bss1_ref, True)
        y = _pool2x2(y, 64, 64, 128).astype(jnp.bfloat16)

        p = jnp.pad(y, ((2, 2), (2, 2), (0, 0)))
        y = _affine(_conv_wmerge(p, w2_ref, 32, 32, 5), bss2_ref, True)
        y = _pool2x2(y, 32, 32, 128).astype(jnp.bfloat16)

        p = jnp.pad(y, ((1, 1), (1, 1), (0, 0)))
        y = _affine(_conv_wmerge(p, w3_ref, 16, 16, 3), bss3_ref, True)
        y = _pool2x2(y, 16, 16, 256).astype(jnp.bfloat16)

        p = jnp.pad(y, ((1, 1), (1, 1), (0, 0)))
        y = _affine(_conv_wmerge(p, w4_ref, 8, 8, 3), bss4_ref, False)
        o_ref[u, 0] = y.astype(jnp.bfloat16)


def _extract(xq, w1m, bss1, w2, bss2, w3, bss3, w4, bss4):
    n = xq.shape[0]
    return pl.pallas_call(
        _extractor_kernel,
        out_shape=jax.ShapeDtypeStruct((2, n, 64, 512), jnp.bfloat16),
        grid=(n,),
        in_specs=[
            pl.BlockSpec((1, 2, 4, 68, 68), lambda b: (b, 0, 0, 0, 0)),
            pl.BlockSpec((100, 128), lambda b: (0, 0)),
            pl.BlockSpec((3, 128), lambda b: (0, 0)),
            pl.BlockSpec((5, 640, 128), lambda b: (0, 0, 0)),
            pl.BlockSpec((3, 128), lambda b: (0, 0)),
            pl.BlockSpec((3, 384, 256), lambda b: (0, 0, 0)),
            pl.BlockSpec((3, 256), lambda b: (0, 0)),
            pl.BlockSpec((3, 768, 512), lambda b: (0, 0, 0)),
            pl.BlockSpec((3, 512), lambda b: (0, 0)),
        ],
        out_specs=pl.BlockSpec((2, 1, 64, 512), lambda b: (0, b, 0, 0)),
        compiler_params=_cp(("parallel",)),
    )(xq, w1m, bss1, w2, bss2, w3, bss3, w4, bss4)


def _fc5_kernel(f_ref, w_ref, bss_ref, o_ref, acc_ref):
    k = pl.program_id(1)

    @pl.when(k == 0)
    def _init():
        acc_ref[...] = jnp.zeros_like(acc_ref)

    acc_ref[...] += jnp.dot(f_ref[...], w_ref[...],
                            preferred_element_type=jnp.float32)

    @pl.when(k == pl.num_programs(1) - 1)
    def _finish():
        o_ref[...] = _affine(acc_ref[...], bss_ref[...], True
                             ).astype(jnp.bfloat16)


def _fc5(flat, fc5_w, fc5_bss, tk=4096, tn=512):
    m, k5 = flat.shape
    n5 = fc5_w.shape[1]
    grid = (n5 // tn, k5 // tk)
    return pl.pallas_call(
        _fc5_kernel,
        out_shape=jax.ShapeDtypeStruct((m, n5), jnp.bfloat16),
        grid=grid,
        in_specs=[
            pl.BlockSpec((m, tk), lambda n, k: (0, k)),
            pl.BlockSpec((tk, tn), lambda n, k: (k, n)),
            pl.BlockSpec((3, tn), lambda n, k: (0, n)),
        ],
        out_specs=pl.BlockSpec((m, tn), lambda n, k: (0, n)),
        scratch_shapes=[pltpu.VMEM((m, tn), jnp.float32)],
        compiler_params=_cp(("parallel", "arbitrary")),
    )(flat, fc5_w, fc5_bss)


def _mlp_kernel(fv_ref, w1_ref, b1_ref, w2_ref, b2_ref, o_ref, *, batch):
    fv = fv_ref[...]
    n5 = fv.shape[1]
    h = jnp.dot(fv[:batch], w1_ref[:n5], preferred_element_type=jnp.float32)
    h = h + jnp.dot(fv[batch:], w1_ref[n5:],
                    preferred_element_type=jnp.float32)
    h = jnp.maximum(h + b1_ref[...], 0.0).astype(jnp.bfloat16)
    o_ref[...] = jnp.dot(h, w2_ref[...],
                         preferred_element_type=jnp.float32) + b2_ref[...]


def _mlp(fv, fc1_w, fc1_b, fc2_w, fc2_b, batch):
    return pl.pallas_call(
        functools.partial(_mlp_kernel, batch=batch),
        out_shape=jax.ShapeDtypeStruct((batch, 2), jnp.float32),
        grid=(1,),
        in_specs=[
            pl.BlockSpec(fv.shape, lambda i: (0, 0)),
            pl.BlockSpec(fc1_w.shape, lambda i: (0, 0)),
            pl.BlockSpec(fc1_b.shape, lambda i: (0, 0)),
            pl.BlockSpec(fc2_w.shape, lambda i: (0, 0)),
            pl.BlockSpec(fc2_b.shape, lambda i: (0, 0)),
        ],
        out_specs=pl.BlockSpec((batch, 2), lambda i: (0, 0)),
        compiler_params=_cp(("arbitrary",)),
    )(fv, fc1_w, fc1_b, fc2_w, fc2_b)


def kernel(x_nchw, conv1_w, conv1_bss, conv2_w, conv2_bss, conv3_w, conv3_bss,
           conv4_w, conv4_bss, fc5_w, fc5_bss, fc1_w, fc1_b, fc2_w, fc2_b):
    B = x_nchw.shape[0]
    xq = jnp.pad(x_nchw.astype(jnp.bfloat16).reshape(B, 2, 3, 64, 64),
                 ((0, 0), (0, 0), (0, 1), (2, 2), (2, 2)))

    w1 = conv1_w.reshape(5, 5, 3, 128)
    w1 = jnp.pad(w1, ((0, 0), (0, 0), (0, 1), (0, 0))).reshape(100, 128)
    w2 = conv2_w.reshape(5, 5 * 128, 128)
    w3 = conv3_w.reshape(3, 3 * 128, 256)
    w4 = conv4_w.reshape(3, 3 * 256, 512)

    feats = _extract(xq, w1, conv1_bss, w2, conv2_bss,
                     w3, conv3_bss, w4, conv4_bss)
    flat = feats.reshape(2 * B, 64 * 512)
    fv = _fc5(flat, fc5_w, fc5_bss)
    return _mlp(fv, fc1_w, fc1_b, fc2_w, fc2_b, B)
